# Initial kernel scaffold; baseline (speedup 1.0000x reference)
import jax
import jax.numpy as jnp
from jax.experimental import pallas as pl


def kernel(x, edge_index, a1_W, a1_b, a2_W, a2_b, s1_W, s1_b, s2_W, s2_b, t1_W, t1_b, t2_W, t2_b, att_W, att_b, dx_W, dx_b, ds_W, ds_b):
    raise NotImplementedError("write your pallas kernel here")


# speedup vs baseline: 7.3886x; 7.3886x over previous
import functools

import jax
import jax.numpy as jnp
from jax import lax
from jax.experimental import pallas as pl
from jax.experimental.pallas import tpu as pltpu
from jax.experimental.pallas import tpu_sc as plsc

NN = 10000
EE = 320000
FF = 128

NC, NS, LANES = 2, 16, 16
NW = NC * NS
NP = 10240
ROWS_PER_TILE = NP // NS
EB = 128
NB = 80
EPW = NB * EB
EPAD = NW * EPW
DUMMY = NN

BN = 1024
GRID_N = NP // BN
BM = 200

_mesh = plsc.VectorSubcoreMesh(core_axis_name="c", subcore_axis_name="s")


def _deg_body(dst_hbm, out_hbm, didx, hist):
    cid = lax.axis_index("c")
    sid = lax.axis_index("s")
    wid = cid * NS + sid
    zero16 = jnp.zeros((LANES,), jnp.float32)

    def zh(i, c):
        hist[pl.ds(i * LANES, LANES)] = zero16
        return c
    lax.fori_loop(0, NP // LANES, zh, 0)

    pltpu.sync_copy(dst_hbm.at[wid], didx)
    one16 = jnp.full((LANES,), 1.0, jnp.float32)

    def blk(b, c):
        for j in range(EB // LANES):
            idx = didx[b, pl.ds(j * LANES, LANES)]
            plsc.addupdate_scatter(hist, [idx], one16)
        return c
    lax.fori_loop(0, NB, blk, 0)
    pltpu.sync_copy(hist, out_hbm.at[wid])


@jax.jit
def _sc_degree(dst3):
    return pl.kernel(
        _deg_body,
        out_type=jax.ShapeDtypeStruct((NW, NP), jnp.float32),
        mesh=_mesh,
        compiler_params=pltpu.CompilerParams(needs_layout_passes=False),
        scratch_types=[
            pltpu.VMEM((NB, EB), jnp.int32),
            pltpu.VMEM((NP,), jnp.float32),
        ],
    )(dst3)


def _scat_body(u_hbm, src_hbm, dst_hbm, out_hbm, sidx, didx, rows, zblk, acc,
               sem):
    cid = lax.axis_index("c")
    sid = lax.axis_index("s")
    wid = cid * NS + sid
    base = sid * ROWS_PER_TILE
    zero16 = jnp.zeros((LANES,), jnp.float32)

    def zrow(r, c):
        for j in range(FF // LANES):
            zblk[r, pl.ds(j * LANES, LANES)] = zero16
        return c
    lax.fori_loop(0, 64, zrow, 0)

    def zcopy(k, c):
        pltpu.sync_copy(zblk, acc.at[pl.ds(base + k * 64, 64)])
        return c
    lax.fori_loop(0, ROWS_PER_TILE // 64, zcopy, 0)

    pltpu.sync_copy(src_hbm.at[wid], sidx)
    pltpu.sync_copy(dst_hbm.at[wid], didx)
    plsc.subcore_barrier()

    def eblk(b, c):
        pltpu.async_copy(u_hbm.at[sidx.at[b]], rows, sem).wait()
        pltpu.sync_copy(rows, acc.at[didx.at[b]], add=True)
        return c
    lax.fori_loop(0, NB, eblk, 0)

    plsc.subcore_barrier()
    pltpu.sync_copy(acc.at[pl.ds(base, ROWS_PER_TILE)],
                    out_hbm.at[cid, pl.ds(base, ROWS_PER_TILE)])


@jax.jit
def _sc_scatter(u, src3, dst3):
    return pl.kernel(
        _scat_body,
        out_type=jax.ShapeDtypeStruct((NC, NP, FF), jnp.float32),
        mesh=_mesh,
        scratch_types=[
            pltpu.VMEM((NB, EB), jnp.int32),
            pltpu.VMEM((NB, EB), jnp.int32),
            pltpu.VMEM((EB, FF), jnp.float32),
            pltpu.VMEM((64, FF), jnp.float32),
            pltpu.VMEM_SHARED((NP, FF), jnp.float32),
            pltpu.SemaphoreType.DMA,
        ],
    )(u, src3, dst3)


def _prep_body(parts_ref, x_ref, dinv_ref, u0_ref):
    deg = jnp.sum(parts_ref[...], axis=0) + 1.0
    dinv = lax.rsqrt(deg)
    dinv_ref[...] = dinv
    u0_ref[...] = x_ref[...] * dinv[:, None]


@jax.jit
def _tc_prep(parts, x):
    return pl.pallas_call(
        _prep_body,
        grid=(GRID_N,),
        in_specs=[
            pl.BlockSpec((NW, BN), lambda i: (0, i)),
            pl.BlockSpec((BN, FF), lambda i: (i, 0)),
        ],
        out_specs=[
            pl.BlockSpec((BN,), lambda i: (i,)),
            pl.BlockSpec((BN, FF), lambda i: (i, 0)),
        ],
        out_shape=[
            jax.ShapeDtypeStruct((NP,), jnp.float32),
            jax.ShapeDtypeStruct((NP, FF), jnp.float32),
        ],
    )(parts, x)


def _l1_body(s_ref, u0_ref, dinv_ref, w_ref, b_ref, o1, o2, o3):
    dinv = dinv_ref[...][:, None]
    p = (s_ref[0] + s_ref[1] + u0_ref[...]) * dinv
    h = jnp.dot(p, w_ref[...], preferred_element_type=jnp.float32) + b_ref[...]
    h = jnp.maximum(h, 0.0) * dinv
    o1[...] = h[:, :FF]
    o2[...] = h[:, FF:2 * FF]
    o3[...] = h[:, 2 * FF:]


@jax.jit
def _tc_l1(s0, u0, dinv, w1, b1):
    blk = pl.BlockSpec((BN, FF), lambda i: (i, 0))
    return pl.pallas_call(
        _l1_body,
        grid=(GRID_N,),
        in_specs=[
            pl.BlockSpec((NC, BN, FF), lambda i: (0, i, 0)),
            blk,
            pl.BlockSpec((BN,), lambda i: (i,)),
            pl.BlockSpec((FF, 3 * FF), lambda i: (0, 0)),
            pl.BlockSpec((1, 3 * FF), lambda i: (0, 0)),
        ],
        out_specs=[blk, blk, blk],
        out_shape=[jax.ShapeDtypeStruct((NP, FF), jnp.float32)] * 3,
    )(s0, u0, dinv, w1, b1)


def _fuse_body(sa_ref, sb_ref, sc_ref, ua_ref, ub_ref, uc_ref, dinv_ref,
               wa_ref, ba_ref, ws_ref, bs_ref, wt_ref, bt_ref,
               aw0_ref, ab0_ref, aw1_ref, ab1_ref, aw2_ref, ab2_ref,
               u2_ref, at0_ref, at1_ref, at2_ref):
    dinv = dinv_ref[...][:, None]

    def head(s_ref, u_ref, w_ref, b_ref):
        q = (s_ref[0] + s_ref[1] + u_ref[...]) * dinv
        h = jnp.dot(q, w_ref[...], preferred_element_type=jnp.float32)
        return jnp.maximum(h + b_ref[...], 0.0)

    ha = head(sa_ref, ua_ref, wa_ref, ba_ref)
    hs = head(sb_ref, ub_ref, ws_ref, bs_ref)
    ht = head(sc_ref, uc_ref, wt_ref, bt_ref)
    hcat = jnp.concatenate([ha, hs, ht], axis=1)
    a0 = jnp.dot(hcat, aw0_ref[...], preferred_element_type=jnp.float32) + ab0_ref[...]
    a1 = jnp.dot(hcat, aw1_ref[...], preferred_element_type=jnp.float32) + ab1_ref[...]
    a2 = jnp.dot(hcat, aw2_ref[...], preferred_element_type=jnp.float32) + ab2_ref[...]
    m = jnp.maximum(jnp.maximum(a0, a1), a2)
    e0 = jnp.exp(a0 - m)
    e1 = jnp.exp(a1 - m)
    e2 = jnp.exp(a2 - m)
    inv = 1.0 / (e0 + e1 + e2)
    t0 = e0 * inv
    t1 = e1 * inv
    t2 = e2 * inv
    h = ha * t0 + hs * t1 + ht * t2
    u2_ref[...] = h * dinv
    at0_ref[...] = t0
    at1_ref[...] = t1
    at2_ref[...] = t2


@jax.jit
def _tc_fuse(sa, sb, sc, ua, ub, uc, dinv, wa, ba, ws, bs, wt, bt,
             aw0, ab0, aw1, ab1, aw2, ab2):
    blk = pl.BlockSpec((BN, FF), lambda i: (i, 0))
    sblk = pl.BlockSpec((NC, BN, FF), lambda i: (0, i, 0))
    wblk = pl.BlockSpec((FF, FF), lambda i: (0, 0))
    bblk = pl.BlockSpec((1, FF), lambda i: (0, 0))
    awblk = pl.BlockSpec((3 * FF, FF), lambda i: (0, 0))
    return pl.pallas_call(
        _fuse_body,
        grid=(GRID_N,),
        in_specs=[
            sblk, sblk, sblk, blk, blk, blk,
            pl.BlockSpec((BN,), lambda i: (i,)),
            wblk, bblk, wblk, bblk, wblk, bblk,
            awblk, bblk, awblk, bblk, awblk, bblk,
        ],
        out_specs=[blk, blk, blk, blk],
        out_shape=[jax.ShapeDtypeStruct((NP, FF), jnp.float32)] * 4,
    )(sa, sb, sc, ua, ub, uc, dinv, wa, ba, ws, bs, wt, bt,
      aw0, ab0, aw1, ab1, aw2, ab2)


def _dec_body(s_ref, u2_ref, dinv_ref, wx_ref, bx_ref, ws_ref, bs_ref,
              x_ref, h_ref):
    dinv = dinv_ref[...][:, None]
    r = (s_ref[0] + s_ref[1] + u2_ref[...]) * dinv
    x_ref[...] = jnp.dot(r, wx_ref[...], preferred_element_type=jnp.float32) + bx_ref[...]
    h_ref[...] = jnp.dot(r, ws_ref[...], preferred_element_type=jnp.float32) + bs_ref[...]


@jax.jit
def _tc_dec(s2, u2, dinv, wx, bx, ws, bs):
    blk = pl.BlockSpec((BN, FF), lambda i: (i, 0))
    return pl.pallas_call(
        _dec_body,
        grid=(GRID_N,),
        in_specs=[
            pl.BlockSpec((NC, BN, FF), lambda i: (0, i, 0)),
            blk,
            pl.BlockSpec((BN,), lambda i: (i,)),
            pl.BlockSpec((FF, FF), lambda i: (0, 0)),
            pl.BlockSpec((1, FF), lambda i: (0, 0)),
            pl.BlockSpec((FF, FF), lambda i: (0, 0)),
            pl.BlockSpec((1, FF), lambda i: (0, 0)),
        ],
        out_specs=[blk, blk],
        out_shape=[jax.ShapeDtypeStruct((NP, FF), jnp.float32)] * 2,
    )(s2, u2, dinv, wx, bx, ws, bs)


def _gram_body(a_ref, b_ref, o_ref):
    o_ref[...] = lax.dot_general(
        a_ref[...], b_ref[...], (((1,), (1,)), ((), ())),
        preferred_element_type=jnp.float32)


@jax.jit
def _tc_gram(h):
    return pl.pallas_call(
        _gram_body,
        grid=(NN // BM,),
        in_specs=[
            pl.BlockSpec((BM, FF), lambda i: (i, 0)),
            pl.BlockSpec((NN, FF), lambda i: (0, 0)),
        ],
        out_specs=pl.BlockSpec((BM, NN), lambda i: (i, 0)),
        out_shape=jax.ShapeDtypeStruct((NN, NN), jnp.float32),
    )(h, h)


def kernel(x, edge_index, a1_W, a1_b, a2_W, a2_b, s1_W, s1_b, s2_W, s2_b,
           t1_W, t1_b, t2_W, t2_b, att_W, att_b, dx_W, dx_b, ds_W, ds_b):
    pad = EPAD - EE
    src3 = jnp.concatenate(
        [edge_index[0], jnp.zeros((pad,), jnp.int32)]).reshape(NW, NB, EB)
    dst3 = jnp.concatenate(
        [edge_index[1], jnp.full((pad,), DUMMY, jnp.int32)]).reshape(NW, NB, EB)

    xp = jnp.pad(x, ((0, NP - NN), (0, 0)))
    deg_parts = _sc_degree(dst3)
    dinv, u0 = _tc_prep(deg_parts, xp)

    s0 = _sc_scatter(u0, src3, dst3)
    w1 = jnp.concatenate([a1_W, s1_W, t1_W], axis=1)
    b1 = jnp.concatenate([a1_b, s1_b, t1_b]).reshape(1, 3 * FF)
    u1a, u1b, u1c = _tc_l1(s0, u0, dinv, w1, b1)

    s1a = _sc_scatter(u1a, src3, dst3)
    s1b = _sc_scatter(u1b, src3, dst3)
    s1c = _sc_scatter(u1c, src3, dst3)

    aw = [att_W[:, k::3] for k in range(3)]
    ab = [att_b[k::3].reshape(1, FF) for k in range(3)]
    u2, at0, at1, at2 = _tc_fuse(
        s1a, s1b, s1c, u1a, u1b, u1c, dinv,
        a2_W, a2_b.reshape(1, FF), s2_W, s2_b.reshape(1, FF),
        t2_W, t2_b.reshape(1, FF),
        aw[0], ab[0], aw[1], ab[1], aw[2], ab[2])

    s2 = _sc_scatter(u2, src3, dst3)
    x_, h_ = _tc_dec(s2, u2, dinv,
                     dx_W, dx_b.reshape(1, FF), ds_W, ds_b.reshape(1, FF))
    s_ = _tc_gram(h_[:NN])
    att = jnp.stack([at0[:NN], at1[:NN], at2[:NN]], axis=-1)
    return (x_[:NN], s_, att)


# speedup vs baseline: 8.3228x; 1.1264x over previous
import functools

import jax
import jax.numpy as jnp
from jax import lax
from jax.experimental import pallas as pl
from jax.experimental.pallas import tpu as pltpu
from jax.experimental.pallas import tpu_sc as plsc

NN = 10000
EE = 320000
FF = 128

NC, NS, LANES = 2, 16, 16
NW = NC * NS
NP = 10240
ROWS_PER_TILE = NP // NS
EB = 128
NB = 80
EPW = NB * EB
EPAD = NW * EPW
DUMMY = NN

BN = 1024
GRID_N = NP // BN
BM = 200

_mesh = plsc.VectorSubcoreMesh(core_axis_name="c", subcore_axis_name="s")


def _deg_body(dst_hbm, out_hbm, didx, hist):
    cid = lax.axis_index("c")
    sid = lax.axis_index("s")
    wid = cid * NS + sid
    zero16 = jnp.zeros((LANES,), jnp.float32)

    def zh(i, c):
        hist[pl.ds(i * LANES, LANES)] = zero16
        return c
    lax.fori_loop(0, NP // LANES, zh, 0)

    pltpu.sync_copy(dst_hbm.at[wid], didx)
    one16 = jnp.full((LANES,), 1.0, jnp.float32)

    def blk(b, c):
        for j in range(EB // LANES):
            idx = didx[b, pl.ds(j * LANES, LANES)]
            plsc.addupdate_scatter(hist, [idx], one16)
        return c
    lax.fori_loop(0, NB, blk, 0)
    pltpu.sync_copy(hist, out_hbm.at[wid])


@jax.jit
def _sc_degree(dst3):
    return pl.kernel(
        _deg_body,
        out_type=jax.ShapeDtypeStruct((NW, NP), jnp.float32),
        mesh=_mesh,
        compiler_params=pltpu.CompilerParams(needs_layout_passes=False),
        scratch_types=[
            pltpu.VMEM((NB, EB), jnp.int32),
            pltpu.VMEM((NP,), jnp.float32),
        ],
    )(dst3)


GCH = 16
NGCH = NB // GCH


def _scat_body(u_hbm, src_hbm, dst_hbm, out_hbm, sidx, didx,
               rows0, rows1, zblk, acc, semr0, semr1, semi0, semi1):
    rows = (rows0, rows1)
    semr = (semr0, semr1)
    semi = (semi0, semi1)
    cid = lax.axis_index("c")
    sid = lax.axis_index("s")
    wid = cid * NS + sid
    base = sid * ROWS_PER_TILE
    zero16 = jnp.zeros((LANES,), jnp.float32)

    def load_idx(ch):
        p = ch & 1
        pltpu.async_copy(src_hbm.at[wid, ch], sidx.at[p], semi[p])
        pltpu.async_copy(dst_hbm.at[wid, ch], didx.at[p], semi[p])

    def wait_idx(ch):
        p = ch & 1
        pltpu.make_async_copy(src_hbm.at[wid, ch], sidx.at[p], semi[p]).wait()
        pltpu.make_async_copy(dst_hbm.at[wid, ch], didx.at[p], semi[p]).wait()

    load_idx(0)
    load_idx(1)

    def zrow(r, c):
        for j in range(FF // LANES):
            zblk[r, pl.ds(j * LANES, LANES)] = zero16
        return c
    lax.fori_loop(0, 32, zrow, 0)

    def zcopy(k, c):
        pltpu.sync_copy(zblk, acc.at[pl.ds(base + k * 32, 32)])
        return c
    lax.fori_loop(0, ROWS_PER_TILE // 32, zcopy, 0)

    wait_idx(0)
    for r in range(2):
        pltpu.async_copy(u_hbm.at[sidx.at[0, r]], rows[r], semr[r])
    plsc.subcore_barrier()

    for b in range(NB):
        ch, j = divmod(b, GCH)
        p = ch & 1
        r = b & 1
        pltpu.make_async_copy(u_hbm.at[sidx.at[p, j]], rows[r],
                              semr[r]).wait()
        pltpu.sync_copy(rows[r], acc.at[didx.at[p, j]], add=True)
        if j == GCH - 1 and ch + 2 < NGCH:
            load_idx(ch + 2)
        nb = b + 2
        if nb < NB:
            nch, nj = divmod(nb, GCH)
            npar = nch & 1
            if nj == 0:
                wait_idx(nch)
            pltpu.async_copy(u_hbm.at[sidx.at[npar, nj]], rows[r], semr[r])

    plsc.subcore_barrier()
    pltpu.sync_copy(acc.at[pl.ds(base, ROWS_PER_TILE)],
                    out_hbm.at[cid, pl.ds(base, ROWS_PER_TILE)])


@jax.jit
def _sc_scatter(u, src4, dst4):
    return pl.kernel(
        _scat_body,
        out_type=jax.ShapeDtypeStruct((NC, NP, FF), jnp.float32),
        mesh=_mesh,
        scratch_types=[
            pltpu.VMEM((2, GCH, EB), jnp.int32),
            pltpu.VMEM((2, GCH, EB), jnp.int32),
            pltpu.VMEM((EB, FF), jnp.float32),
            pltpu.VMEM((EB, FF), jnp.float32),
            pltpu.VMEM((32, FF), jnp.float32),
            pltpu.VMEM_SHARED((NP, FF), jnp.float32),
            pltpu.SemaphoreType.DMA,
            pltpu.SemaphoreType.DMA,
            pltpu.SemaphoreType.DMA,
            pltpu.SemaphoreType.DMA,
        ],
    )(u, src4, dst4)


def _prep_body(parts_ref, x_ref, dinv_ref, u0_ref):
    deg = jnp.sum(parts_ref[...], axis=0) + 1.0
    dinv = lax.rsqrt(deg)
    dinv_ref[...] = dinv
    u0_ref[...] = x_ref[...] * dinv[:, None]


@jax.jit
def _tc_prep(parts, x):
    return pl.pallas_call(
        _prep_body,
        grid=(GRID_N,),
        in_specs=[
            pl.BlockSpec((NW, BN), lambda i: (0, i)),
            pl.BlockSpec((BN, FF), lambda i: (i, 0)),
        ],
        out_specs=[
            pl.BlockSpec((BN,), lambda i: (i,)),
            pl.BlockSpec((BN, FF), lambda i: (i, 0)),
        ],
        out_shape=[
            jax.ShapeDtypeStruct((NP,), jnp.float32),
            jax.ShapeDtypeStruct((NP, FF), jnp.float32),
        ],
    )(parts, x)


def _l1_body(s_ref, u0_ref, dinv_ref, w_ref, b_ref, o1, o2, o3):
    dinv = dinv_ref[...][:, None]
    p = (s_ref[0] + s_ref[1] + u0_ref[...]) * dinv
    h = jnp.dot(p, w_ref[...], preferred_element_type=jnp.float32) + b_ref[...]
    h = jnp.maximum(h, 0.0) * dinv
    o1[...] = h[:, :FF]
    o2[...] = h[:, FF:2 * FF]
    o3[...] = h[:, 2 * FF:]


@jax.jit
def _tc_l1(s0, u0, dinv, w1, b1):
    blk = pl.BlockSpec((BN, FF), lambda i: (i, 0))
    return pl.pallas_call(
        _l1_body,
        grid=(GRID_N,),
        in_specs=[
            pl.BlockSpec((NC, BN, FF), lambda i: (0, i, 0)),
            blk,
            pl.BlockSpec((BN,), lambda i: (i,)),
            pl.BlockSpec((FF, 3 * FF), lambda i: (0, 0)),
            pl.BlockSpec((1, 3 * FF), lambda i: (0, 0)),
        ],
        out_specs=[blk, blk, blk],
        out_shape=[jax.ShapeDtypeStruct((NP, FF), jnp.float32)] * 3,
    )(s0, u0, dinv, w1, b1)


def _fuse_body(sa_ref, sb_ref, sc_ref, ua_ref, ub_ref, uc_ref, dinv_ref,
               wa_ref, ba_ref, ws_ref, bs_ref, wt_ref, bt_ref,
               aw0_ref, ab0_ref, aw1_ref, ab1_ref, aw2_ref, ab2_ref,
               u2_ref, at0_ref, at1_ref, at2_ref):
    dinv = dinv_ref[...][:, None]

    def head(s_ref, u_ref, w_ref, b_ref):
        q = (s_ref[0] + s_ref[1] + u_ref[...]) * dinv
        h = jnp.dot(q, w_ref[...], preferred_element_type=jnp.float32)
        return jnp.maximum(h + b_ref[...], 0.0)

    ha = head(sa_ref, ua_ref, wa_ref, ba_ref)
    hs = head(sb_ref, ub_ref, ws_ref, bs_ref)
    ht = head(sc_ref, uc_ref, wt_ref, bt_ref)
    hcat = jnp.concatenate([ha, hs, ht], axis=1)
    a0 = jnp.dot(hcat, aw0_ref[...], preferred_element_type=jnp.float32) + ab0_ref[...]
    a1 = jnp.dot(hcat, aw1_ref[...], preferred_element_type=jnp.float32) + ab1_ref[...]
    a2 = jnp.dot(hcat, aw2_ref[...], preferred_element_type=jnp.float32) + ab2_ref[...]
    m = jnp.maximum(jnp.maximum(a0, a1), a2)
    e0 = jnp.exp(a0 - m)
    e1 = jnp.exp(a1 - m)
    e2 = jnp.exp(a2 - m)
    inv = 1.0 / (e0 + e1 + e2)
    t0 = e0 * inv
    t1 = e1 * inv
    t2 = e2 * inv
    h = ha * t0 + hs * t1 + ht * t2
    u2_ref[...] = h * dinv
    at0_ref[...] = t0
    at1_ref[...] = t1
    at2_ref[...] = t2


@jax.jit
def _tc_fuse(sa, sb, sc, ua, ub, uc, dinv, wa, ba, ws, bs, wt, bt,
             aw0, ab0, aw1, ab1, aw2, ab2):
    blk = pl.BlockSpec((BN, FF), lambda i: (i, 0))
    sblk = pl.BlockSpec((NC, BN, FF), lambda i: (0, i, 0))
    wblk = pl.BlockSpec((FF, FF), lambda i: (0, 0))
    bblk = pl.BlockSpec((1, FF), lambda i: (0, 0))
    awblk = pl.BlockSpec((3 * FF, FF), lambda i: (0, 0))
    return pl.pallas_call(
        _fuse_body,
        grid=(GRID_N,),
        in_specs=[
            sblk, sblk, sblk, blk, blk, blk,
            pl.BlockSpec((BN,), lambda i: (i,)),
            wblk, bblk, wblk, bblk, wblk, bblk,
            awblk, bblk, awblk, bblk, awblk, bblk,
        ],
        out_specs=[blk, blk, blk, blk],
        out_shape=[jax.ShapeDtypeStruct((NP, FF), jnp.float32)] * 4,
    )(sa, sb, sc, ua, ub, uc, dinv, wa, ba, ws, bs, wt, bt,
      aw0, ab0, aw1, ab1, aw2, ab2)


def _dec_body(s_ref, u2_ref, dinv_ref, wx_ref, bx_ref, ws_ref, bs_ref,
              x_ref, h_ref):
    dinv = dinv_ref[...][:, None]
    r = (s_ref[0] + s_ref[1] + u2_ref[...]) * dinv
    x_ref[...] = jnp.dot(r, wx_ref[...], preferred_element_type=jnp.float32) + bx_ref[...]
    h_ref[...] = jnp.dot(r, ws_ref[...], preferred_element_type=jnp.float32) + bs_ref[...]


@jax.jit
def _tc_dec(s2, u2, dinv, wx, bx, ws, bs):
    blk = pl.BlockSpec((BN, FF), lambda i: (i, 0))
    return pl.pallas_call(
        _dec_body,
        grid=(GRID_N,),
        in_specs=[
            pl.BlockSpec((NC, BN, FF), lambda i: (0, i, 0)),
            blk,
            pl.BlockSpec((BN,), lambda i: (i,)),
            pl.BlockSpec((FF, FF), lambda i: (0, 0)),
            pl.BlockSpec((1, FF), lambda i: (0, 0)),
            pl.BlockSpec((FF, FF), lambda i: (0, 0)),
            pl.BlockSpec((1, FF), lambda i: (0, 0)),
        ],
        out_specs=[blk, blk],
        out_shape=[jax.ShapeDtypeStruct((NP, FF), jnp.float32)] * 2,
    )(s2, u2, dinv, wx, bx, ws, bs)


def _gram_body(a_ref, b_ref, o_ref):
    o_ref[...] = lax.dot_general(
        a_ref[...], b_ref[...], (((1,), (1,)), ((), ())),
        preferred_element_type=jnp.float32)


@jax.jit
def _tc_gram(h):
    return pl.pallas_call(
        _gram_body,
        grid=(NN // BM,),
        in_specs=[
            pl.BlockSpec((BM, FF), lambda i: (i, 0)),
            pl.BlockSpec((NN, FF), lambda i: (0, 0)),
        ],
        out_specs=pl.BlockSpec((BM, NN), lambda i: (i, 0)),
        out_shape=jax.ShapeDtypeStruct((NN, NN), jnp.float32),
    )(h, h)


def kernel(x, edge_index, a1_W, a1_b, a2_W, a2_b, s1_W, s1_b, s2_W, s2_b,
           t1_W, t1_b, t2_W, t2_b, att_W, att_b, dx_W, dx_b, ds_W, ds_b):
    pad = EPAD - EE
    src3 = jnp.concatenate(
        [edge_index[0], jnp.zeros((pad,), jnp.int32)]).reshape(NW, NB, EB)
    dst3 = jnp.concatenate(
        [edge_index[1], jnp.full((pad,), DUMMY, jnp.int32)]).reshape(NW, NB, EB)
    src4 = src3.reshape(NW, NGCH, GCH, EB)
    dst4 = dst3.reshape(NW, NGCH, GCH, EB)

    xp = jnp.pad(x, ((0, NP - NN), (0, 0)))
    deg_parts = _sc_degree(dst3)
    dinv, u0 = _tc_prep(deg_parts, xp)

    s0 = _sc_scatter(u0, src4, dst4)
    w1 = jnp.concatenate([a1_W, s1_W, t1_W], axis=1)
    b1 = jnp.concatenate([a1_b, s1_b, t1_b]).reshape(1, 3 * FF)
    u1a, u1b, u1c = _tc_l1(s0, u0, dinv, w1, b1)

    s1a = _sc_scatter(u1a, src4, dst4)
    s1b = _sc_scatter(u1b, src4, dst4)
    s1c = _sc_scatter(u1c, src4, dst4)

    aw = [att_W[:, k::3] for k in range(3)]
    ab = [att_b[k::3].reshape(1, FF) for k in range(3)]
    u2, at0, at1, at2 = _tc_fuse(
        s1a, s1b, s1c, u1a, u1b, u1c, dinv,
        a2_W, a2_b.reshape(1, FF), s2_W, s2_b.reshape(1, FF),
        t2_W, t2_b.reshape(1, FF),
        aw[0], ab[0], aw[1], ab[1], aw[2], ab[2])

    s2 = _sc_scatter(u2, src4, dst4)
    x_, h_ = _tc_dec(s2, u2, dinv,
                     dx_W, dx_b.reshape(1, FF), ds_W, ds_b.reshape(1, FF))
    s_ = _tc_gram(h_[:NN])
    att = jnp.stack([at0[:NN], at1[:NN], at2[:NN]], axis=-1)
    return (x_[:NN], s_, att)
